# baseline (speedup 1.0000x reference)
import functools

import jax
import jax.numpy as jnp
from jax import lax
from jax.experimental import pallas as pl
from jax.experimental.pallas import tpu as pltpu
from jax.experimental.pallas import tpu_sc as plsc

_LANES = 16


def _make_sc_kernel(B, S, D, n_workers, chunk):
    assert S % n_workers == 0
    s_per_w = S // n_workers
    assert s_per_w % chunk == 0
    n_chunks = s_per_w // chunk
    chunk_elems = chunk * D

    mesh = plsc.VectorSubcoreMesh(core_axis_name="c", subcore_axis_name="s")
    num_cores = mesh.num_cores

    @functools.partial(
        pl.kernel,
        out_type=jax.ShapeDtypeStruct((B * S * D,), jnp.float32),
        mesh=mesh,
        scratch_types=[
            pltpu.VMEM((chunk_elems,), jnp.float32),
            pltpu.VMEM((chunk_elems,), jnp.float32),
        ],
    )
    def sc_add(x_hbm, pos_hbm, out_hbm, pos_v, xb_v):
        wid = lax.axis_index("s") * num_cores + lax.axis_index("c")

        def chunk_body(c, _):
            s0 = wid * s_per_w + c * chunk
            pltpu.sync_copy(pos_hbm.at[pl.ds(s0 * D, chunk_elems)], pos_v)

            for b in range(B):
                base = (b * S + s0) * D
                pltpu.sync_copy(x_hbm.at[pl.ds(base, chunk_elems)], xb_v)

                def add_body(i, _):
                    sl = pl.ds(i * _LANES, _LANES)
                    xb_v[sl] = xb_v[sl] + pos_v[sl]
                    return 0

                lax.fori_loop(0, chunk_elems // _LANES, add_body, 0)
                pltpu.sync_copy(xb_v, out_hbm.at[pl.ds(base, chunk_elems)])
            return 0

        lax.fori_loop(0, n_chunks, chunk_body, 0)

    return sc_add


def kernel(x, pos_table):
    B, S, D = x.shape
    sc_add = _make_sc_kernel(B, S, D, n_workers=32, chunk=8)
    out_flat = sc_add(x.reshape(-1), pos_table.reshape(-1))
    return out_flat.reshape(B, S, D)


# speedup vs baseline: 1.6652x; 1.6652x over previous
import functools

import jax
import jax.numpy as jnp
from jax import lax
from jax.experimental import pallas as pl
from jax.experimental.pallas import tpu as pltpu
from jax.experimental.pallas import tpu_sc as plsc

_LANES = 16


def _make_sc_kernel(B, S, D, n_workers, chunk):
    assert S % n_workers == 0
    s_per_w = S // n_workers
    assert s_per_w % chunk == 0
    n_chunks = s_per_w // chunk
    assert n_chunks % 2 == 0 and n_chunks >= 4
    ce = chunk * D

    mesh = plsc.VectorSubcoreMesh(core_axis_name="c", subcore_axis_name="s")
    num_cores = mesh.num_cores

    @functools.partial(
        pl.kernel,
        out_type=jax.ShapeDtypeStruct((B * S * D,), jnp.float32),
        mesh=mesh,
        scratch_types=[
            [pltpu.VMEM((ce,), jnp.float32) for _ in range(2)],
            [pltpu.VMEM((ce,), jnp.float32) for _ in range(2)],
            [pltpu.VMEM((ce,), jnp.float32) for _ in range(2)],
            [pltpu.SemaphoreType.DMA for _ in range(2)],
            [pltpu.SemaphoreType.DMA for _ in range(2)],
            [pltpu.SemaphoreType.DMA for _ in range(2)],
        ],
    )
    def sc_add(x_hbm, pos_hbm, out_hbm, pos_v, x_v, o_v, psem, xsem, osem):
        wid = lax.axis_index("s") * num_cores + lax.axis_index("c")
        base_s = wid * s_per_w

        def pos_src(c):
            return pos_hbm.at[pl.ds((base_s + c * chunk) * D, ce)]

        def x_src(c, b):
            return x_hbm.at[pl.ds((b * S + base_s + c * chunk) * D, ce)]

        def out_dst(c, b):
            return out_hbm.at[pl.ds((b * S + base_s + c * chunk) * D, ce)]

        pltpu.make_async_copy(pos_src(0), pos_v[0], psem[0]).start()
        pltpu.make_async_copy(x_src(0, 0), x_v[0], xsem[0]).start()

        @pl.loop(0, n_chunks, step=2)
        def chunk_pair(cpair):
            for cc in range(2):
                c = cpair + cc
                pltpu.make_async_copy(pos_src(c), pos_v[cc], psem[cc]).wait()

                @pl.when(c + 1 < n_chunks)
                def _():
                    pltpu.make_async_copy(
                        pos_src(c + 1), pos_v[1 - cc], psem[1 - cc]
                    ).start()

                for b in range(B):
                    sl = b % 2
                    nsl = (b + 1) % 2
                    pltpu.make_async_copy(x_src(c, b), x_v[sl], xsem[sl]).wait()

                    if b + 1 < B:
                        pltpu.make_async_copy(
                            x_src(c, b + 1), x_v[nsl], xsem[nsl]
                        ).start()
                    else:

                        @pl.when(c + 1 < n_chunks)
                        def _():
                            pltpu.make_async_copy(
                                x_src(c + 1, 0), x_v[nsl], xsem[nsl]
                            ).start()

                    if b < 2:

                        @pl.when(c >= 1)
                        def _():
                            pltpu.make_async_copy(
                                o_v[sl], out_dst(c - 1, b + 2), osem[sl]
                            ).wait()

                    else:
                        pltpu.make_async_copy(
                            o_v[sl], out_dst(c, b - 2), osem[sl]
                        ).wait()

                    @plsc.parallel_loop(0, ce, step=_LANES, unroll=8)
                    def add_body(i):
                        v = pl.ds(i, _LANES)
                        o_v[sl][v] = x_v[sl][v] + pos_v[cc][v]

                    pltpu.make_async_copy(o_v[sl], out_dst(c, b), osem[sl]).start()

        pltpu.make_async_copy(o_v[(B - 2) % 2], out_dst(n_chunks - 1, B - 2), osem[(B - 2) % 2]).wait()
        pltpu.make_async_copy(o_v[(B - 1) % 2], out_dst(n_chunks - 1, B - 1), osem[(B - 1) % 2]).wait()

    return sc_add


def kernel(x, pos_table):
    B, S, D = x.shape
    sc_add = _make_sc_kernel(B, S, D, n_workers=32, chunk=4)
    out_flat = sc_add(x.reshape(-1), pos_table.reshape(-1))
    return out_flat.reshape(B, S, D)


# speedup vs baseline: 6.0110x; 3.6097x over previous
import functools

import jax
import jax.numpy as jnp
from jax import lax
from jax.experimental import pallas as pl
from jax.experimental.pallas import tpu as pltpu
from jax.experimental.pallas import tpu_sc as plsc

_LANES = 16


def _make_sc_kernel(B, S, D, n_workers):
    rows = 8
    dh = D // 2
    assert S % (n_workers * rows) == 0
    n_sblk = S // (n_workers * rows)
    n_combo = n_sblk * 2

    mesh = plsc.VectorSubcoreMesh(core_axis_name="c", subcore_axis_name="s")
    num_cores = mesh.num_cores

    @functools.partial(
        pl.kernel,
        out_type=jax.ShapeDtypeStruct((B, S, D), jnp.float32),
        mesh=mesh,
        scratch_types=[
            [pltpu.VMEM((rows, dh), jnp.float32) for _ in range(2)],
            [pltpu.VMEM((rows, dh), jnp.float32) for _ in range(B)],
            [pltpu.SemaphoreType.DMA for _ in range(2)],
            [pltpu.SemaphoreType.DMA for _ in range(B)],
            [pltpu.SemaphoreType.DMA for _ in range(B)],
        ],
    )
    def sc_add(x_hbm, pos_hbm, out_hbm, pos_v, x_v, psem, xsem, osem):
        wid = lax.axis_index("s") * num_cores + lax.axis_index("c")
        base_s = wid * n_sblk * rows

        def s0(c):
            return base_s + c * rows

        def pos_cp(c, h, slot):
            return pltpu.make_async_copy(
                pos_hbm.at[pl.ds(s0(c), rows), pl.ds(h * dh, dh)],
                pos_v[slot],
                psem[slot],
            )

        def x_cp(c, h, b):
            return pltpu.make_async_copy(
                x_hbm.at[b, pl.ds(s0(c), rows), pl.ds(h * dh, dh)],
                x_v[b],
                xsem[b],
            )

        def o_cp(c, h, b):
            return pltpu.make_async_copy(
                x_v[b],
                out_hbm.at[b, pl.ds(s0(c), rows), pl.ds(h * dh, dh)],
                osem[b],
            )

        pos_cp(0, 0, 0).start()
        x_cp(0, 0, 0).start()
        x_cp(0, 0, 1).start()

        @pl.loop(0, n_sblk)
        def sblk_loop(c):
            for h in range(2):
                for b in range(B):
                    if b == 0:
                        pos_cp(c, h, h).wait()
                        if h == 0:
                            pos_cp(c, 1, 1).start()
                        else:

                            @pl.when(c + 1 < n_sblk)
                            def _():
                                pos_cp(c + 1, 0, 0).start()

                    x_cp(c, h, b).wait()

                    s2 = (b + 2) % B
                    if b < 2:
                        if h == 1:
                            o_cp(c, 0, s2).wait()
                        else:

                            @pl.when(c >= 1)
                            def _():
                                o_cp(c - 1, 1, s2).wait()

                        x_cp(c, h, s2).start()
                    else:
                        o_cp(c, h, s2).wait()
                        if h == 0:
                            x_cp(c, 1, s2).start()
                        else:

                            @pl.when(c + 1 < n_sblk)
                            def _():
                                x_cp(c + 1, 0, s2).start()

                    for r in range(rows):

                        @plsc.parallel_loop(0, dh, step=_LANES, unroll=8)
                        def add_body(i):
                            v = pl.ds(i, _LANES)
                            x_v[b][r, v] = x_v[b][r, v] + pos_v[h][r, v]

                    o_cp(c, h, b).start()

        for b in range(B - 2, B):
            o_cp(n_sblk - 1, 1, b).wait()

    return sc_add


def kernel(x, pos_table):
    B, S, D = x.shape
    sc_add = _make_sc_kernel(B, S, D, n_workers=32)
    return sc_add(x, pos_table)
